# baseline (speedup 1.0000x reference)
import functools

import jax
import jax.numpy as jnp
from jax import lax
from jax.experimental import pallas as pl
from jax.experimental.pallas import tpu as pltpu
from jax.experimental.pallas import tpu_sc as plsc

D = 32
V = 1000001
VC = 1000448
B = 16384
L = 50
NC = 2
NS = 16
NW = NC * NS

Q = 512
NFULL = 999424 // Q
BLKW = NFULL // NW

BPW = B // NW
CH = 32
ROWS = CH * L
NCHUNK = BPW // CH
G = 80
NG = ROWS // G


def _depad_body(tpad_hbm, out_hbm, in_v, out_v):
    wid = lax.axis_index("s") * NC + lax.axis_index("c")

    def compact(nrows, r0):
        def row_body(r, carry):
            out_v[pl.ds(r * D, 16)] = in_v[r, pl.ds(0, 16)]
            out_v[pl.ds(r * D + 16, 16)] = in_v[r, pl.ds(16, 16)]
            return carry
        lax.fori_loop(0, nrows, row_body, 0)
        pltpu.sync_copy(out_v.at[pl.ds(0, nrows * D)],
                        out_hbm.at[pl.ds(r0 * D, nrows * D)])

    def blk_body(k, carry):
        blk = wid * BLKW + k
        r0 = pl.multiple_of(blk * Q, 8)
        pltpu.sync_copy(tpad_hbm.at[pl.ds(r0, Q)], in_v)
        compact(Q, r0)
        return carry

    lax.fori_loop(0, BLKW, blk_body, 0)

    @pl.when(wid == NW - 1)
    def _tail():
        pltpu.sync_copy(tpad_hbm.at[pl.ds(999424, 512)], in_v)
        compact(512, 999424)
        pltpu.sync_copy(tpad_hbm.at[pl.ds(999992, 8)], in_v.at[pl.ds(0, 8)])
        compact(8, 999992)


@jax.jit
def _depad(table):
    mesh = plsc.VectorSubcoreMesh(core_axis_name="c", subcore_axis_name="s")
    return pl.kernel(
        _depad_body,
        out_type=jax.ShapeDtypeStruct((VC * D,), jnp.float32),
        mesh=mesh,
        compiler_params=pltpu.CompilerParams(use_tc_tiling_on_sc=True),
        scratch_types=[
            pltpu.VMEM((Q, D), jnp.float32),
            pltpu.VMEM((Q * D,), jnp.float32),
        ],
    )(table)


def _embed_bag_body(idx_hbm, len_hbm, table_hbm, out_hbm,
                    idx_v, buf_v, out_v, len_v, sem):
    wid = lax.axis_index("s") * NC + lax.axis_index("c")
    base_b = wid * BPW

    pltpu.sync_copy(len_hbm.at[pl.ds(base_b * 1, BPW)], len_v.at[pl.ds(0, BPW)])

    def chunk_body(c, carry):
        flat_base = pl.multiple_of((base_b + c * CH) * L, 8)
        pltpu.sync_copy(idx_hbm.at[pl.ds(flat_base, ROWS)], idx_v)

        copies = []
        for j in range(NG):
            copies.append(pltpu.async_copy(
                table_hbm.at[idx_v.at[pl.ds(j * G, G)]],
                buf_v.at[pl.ds(j * G, G)],
                sem))
        for cp in copies:
            cp.wait()

        def batch_body(b, bcarry):
            r0 = b * L
            acc0 = buf_v[r0, pl.ds(0, 16)]
            acc1 = buf_v[r0, pl.ds(16, 16)]
            for l in range(1, L):
                acc0 = acc0 + buf_v[r0 + l, pl.ds(0, 16)]
                acc1 = acc1 + buf_v[r0 + l, pl.ds(16, 16)]
            lnv = len_v[pl.ds(c * CH + b, 16)]
            lf = jnp.maximum(lnv[0], 1).astype(jnp.float32)
            out_v[b, pl.ds(0, 16)] = acc0 / lf
            out_v[b, pl.ds(16, 16)] = acc1 / lf
            return bcarry

        lax.fori_loop(0, CH, batch_body, 0)

        out_base = pl.multiple_of(base_b + c * CH, 8)
        pltpu.sync_copy(out_v, out_hbm.at[pl.ds(out_base, CH)])
        return carry

    lax.fori_loop(0, NCHUNK, chunk_body, 0)


@jax.jit
def _embed_bag(idx_flat, len_flat, table_rm):
    mesh = plsc.VectorSubcoreMesh(core_axis_name="c", subcore_axis_name="s")
    return pl.kernel(
        _embed_bag_body,
        out_type=jax.ShapeDtypeStruct((B, D), jnp.float32),
        mesh=mesh,
        compiler_params=pltpu.CompilerParams(use_tc_tiling_on_sc=False),
        scratch_types=[
            pltpu.VMEM((ROWS,), jnp.int32),
            pltpu.VMEM((ROWS, D), jnp.float32),
            pltpu.VMEM((CH, D), jnp.float32),
            pltpu.VMEM((BPW + 16,), jnp.int32),
            pltpu.SemaphoreType.DMA,
        ],
    )(idx_flat, len_flat, table_rm)


def kernel(kw_indices, kw_lengths, embedding_weight):
    idx_flat = kw_indices.reshape(-1).astype(jnp.int32)
    len_flat = kw_lengths.reshape(-1).astype(jnp.int32)
    table_rm = _depad(embedding_weight).reshape(VC, D)
    return _embed_bag(idx_flat, len_flat, table_rm)


# speedup vs baseline: 1.4805x; 1.4805x over previous
import functools

import jax
import jax.numpy as jnp
from jax import lax
from jax.experimental import pallas as pl
from jax.experimental.pallas import tpu as pltpu
from jax.experimental.pallas import tpu_sc as plsc

D = 32
B = 16384
L = 50
NC = 2
NS = 16
NW = NC * NS
BPW = B // NW
CH = 32
ROWS = CH * L
NCHUNK = BPW // CH
G = 80
NG = ROWS // G


def _embed_bag_body(idx_hbm, len_hbm, table_hbm, out_hbm,
                    idx0_v, idx1_v, buf0_v, buf1_v, out_v, len_v, sem0, sem1):
    wid = lax.axis_index("s") * NC + lax.axis_index("c")
    base_b = wid * BPW

    pltpu.sync_copy(len_hbm.at[pl.ds(base_b * 1, BPW)], len_v.at[pl.ds(0, BPW)])

    def fire(c, idx_v, buf_v, sem):
        flat_base = pl.multiple_of((base_b + c * CH) * L, 8)
        pltpu.sync_copy(idx_hbm.at[pl.ds(flat_base, ROWS)], idx_v)
        copies = []
        for j in range(NG):
            copies.append(pltpu.async_copy(
                table_hbm.at[idx_v.at[pl.ds(j * G, G)]],
                buf_v.at[pl.ds(j * G, G)],
                sem))
        return copies

    def accumulate(c, buf_v):
        def batch_body(h, bcarry):
            for s in range(2):
                b = h * 2 + s
                r0 = b * L
                acc0 = buf_v[r0, pl.ds(0, 16)]
                acc1 = buf_v[r0, pl.ds(16, 16)]
                for l in range(1, L):
                    acc0 = acc0 + buf_v[r0 + l, pl.ds(0, 16)]
                    acc1 = acc1 + buf_v[r0 + l, pl.ds(16, 16)]
                lnv = len_v[pl.ds(c * CH + b, 16)]
                lf = jnp.maximum(lnv[0], 1).astype(jnp.float32)
                out_v[b, pl.ds(0, 16)] = acc0 / lf
                out_v[b, pl.ds(16, 16)] = acc1 / lf
            return bcarry

        lax.fori_loop(0, CH // 2, batch_body, 0)
        out_base = pl.multiple_of(base_b + c * CH, 8)
        pltpu.sync_copy(out_v, out_hbm.at[pl.ds(out_base, CH)])

    def pair_body(h, carry):
        c0 = h * 2
        copies0 = fire(c0, idx0_v, buf0_v, sem0)
        copies1 = fire(c0 + 1, idx1_v, buf1_v, sem1)
        for cp in copies0:
            cp.wait()
        accumulate(c0, buf0_v)
        for cp in copies1:
            cp.wait()
        accumulate(c0 + 1, buf1_v)
        return carry

    lax.fori_loop(0, NCHUNK // 2, pair_body, 0)


@jax.jit
def _embed_bag(idx_flat, len_flat, table):
    mesh = plsc.VectorSubcoreMesh(core_axis_name="c", subcore_axis_name="s")
    return pl.kernel(
        _embed_bag_body,
        out_type=jax.ShapeDtypeStruct((B, D), jnp.float32),
        mesh=mesh,
        compiler_params=pltpu.CompilerParams(use_tc_tiling_on_sc=False),
        scratch_types=[
            pltpu.VMEM((ROWS,), jnp.int32),
            pltpu.VMEM((ROWS,), jnp.int32),
            pltpu.VMEM((ROWS, D), jnp.float32),
            pltpu.VMEM((ROWS, D), jnp.float32),
            pltpu.VMEM((CH, D), jnp.float32),
            pltpu.VMEM((BPW + 16,), jnp.int32),
            pltpu.SemaphoreType.DMA,
            pltpu.SemaphoreType.DMA,
        ],
    )(idx_flat, len_flat, table)


def kernel(kw_indices, kw_lengths, embedding_weight):
    idx_flat = kw_indices.reshape(-1).astype(jnp.int32)
    len_flat = kw_lengths.reshape(-1).astype(jnp.int32)
    return _embed_bag(idx_flat, len_flat, embedding_weight)
